# Initial kernel scaffold; baseline (speedup 1.0000x reference)
import jax
import jax.numpy as jnp
from jax.experimental import pallas as pl


def kernel(predicted_locs, predicted_scores, boxes, labels, priors_cxcy):
    raise NotImplementedError("write your pallas kernel here")


# speedup vs baseline: 99.9219x; 99.9219x over previous
import jax
import jax.numpy as jnp
from jax import lax
from jax.experimental import pallas as pl
from jax.experimental.pallas import tpu as pltpu

_B, _C, _P, _X = 8, 8, 8732, 16
_THRESHOLD = 0.5
_NEG_POS_RATIO = 3
_ALPHA = 1.0
_FLT_INF_BITS = 0x7F800000


def _mbl_kernel(pl_ref, ps_ref, bx_ref, lb_ref, pr_ref, out_ref, ce_bits):
    c = pl.program_id(0)
    f32 = jnp.float32

    @pl.when(c == 0)
    def _init():
        out_ref[:, :] = jnp.zeros((1, 1), jnp.float32)

    pcx = pr_ref[0:1, :]
    pcy = pr_ref[1:2, :]
    pw = pr_ref[2:3, :]
    ph = pr_ref[3:4, :]
    px1 = pcx - pw / 2.0
    py1 = pcy - ph / 2.0
    px2 = pcx + pw / 2.0
    py2 = pcy + ph / 2.0
    parea = (px2 - px1) * (py2 - py1)

    bx1 = bx_ref[0, 0]
    by1 = bx_ref[0, 1]
    bx2 = bx_ref[0, 2]
    by2 = bx_ref[0, 3]
    barea = (bx2 - bx1) * (by2 - by1)
    lbv = lb_ref[0]

    lane = lax.broadcasted_iota(jnp.int32, (_B, _P), 1)

    ofp = None
    oidx = jnp.zeros((_B, _P), jnp.int32)
    pfo = []
    for i in range(_X):
        a_x1 = bx1[:, i : i + 1]
        a_y1 = by1[:, i : i + 1]
        a_x2 = bx2[:, i : i + 1]
        a_y2 = by2[:, i : i + 1]
        wx = jnp.maximum(jnp.minimum(a_x2, px2) - jnp.maximum(a_x1, px1), 0.0)
        wy = jnp.maximum(jnp.minimum(a_y2, py2) - jnp.maximum(a_y1, py1), 0.0)
        inter = wx * wy
        ov = inter / (barea[:, i : i + 1] + parea - inter)
        if i == 0:
            ofp = ov
        else:
            upd = ov > ofp
            oidx = jnp.where(upd, i, oidx)
            ofp = jnp.maximum(ofp, ov)
        m = jnp.max(ov, axis=1, keepdims=True)
        pidx = jnp.min(jnp.where(ov == m, lane, _P), axis=1, keepdims=True)
        pfo.append(pidx)

    for i in range(_X):
        msk = lane == pfo[i]
        ofp = jnp.where(msk, 1.0, ofp)
        oidx = jnp.where(msk, i, oidx)

    gx1 = jnp.zeros((_B, _P), f32)
    gy1 = jnp.zeros((_B, _P), f32)
    gx2 = jnp.zeros((_B, _P), f32)
    gy2 = jnp.zeros((_B, _P), f32)
    lab = jnp.zeros((_B, _P), jnp.int32)
    for i in range(_X):
        sel = oidx == i
        gx1 = jnp.where(sel, bx1[:, i : i + 1], gx1)
        gy1 = jnp.where(sel, by1[:, i : i + 1], gy1)
        gx2 = jnp.where(sel, bx2[:, i : i + 1], gx2)
        gy2 = jnp.where(sel, by2[:, i : i + 1], gy2)
        lab = jnp.where(sel, lbv[:, i : i + 1], lab)

    tcls = jnp.where(ofp < _THRESHOLD, 0, lab)
    pos = tcls != 0
    posf = pos.astype(f32)

    gcx = (gx2 + gx1) / 2.0
    gcy = (gy2 + gy1) / 2.0
    gw = gx2 - gx1
    gh = gy2 - gy1
    t0 = (gcx - pcx) / (pw / 10.0)
    t1 = (gcy - pcy) / (ph / 10.0)
    t2 = jnp.log(gw / pw) * 5.0
    t3 = jnp.log(gh / ph) * 5.0

    n_pos_row = jnp.sum(pos.astype(jnp.int32), axis=1, keepdims=True)
    n_pos = jnp.sum(posf)
    n_pos_safe = jnp.maximum(n_pos, 1.0)

    loc_sum = (
        jnp.sum(jnp.abs(pl_ref[0, 0] - t0) * posf)
        + jnp.sum(jnp.abs(pl_ref[0, 1] - t1) * posf)
        + jnp.sum(jnp.abs(pl_ref[0, 2] - t2) * posf)
        + jnp.sum(jnp.abs(pl_ref[0, 3] - t3) * posf)
    )
    loc_loss = loc_sum / (n_pos_safe * 4.0)

    s0 = ps_ref[0, 0]
    s1 = ps_ref[0, 1]
    mx = jnp.maximum(s0, s1)
    lse = mx + jnp.log(jnp.exp(s0 - mx) + jnp.exp(s1 - mx))
    ce = lse - jnp.where(tcls == 1, s1, s0)
    pos_sum = jnp.sum(jnp.where(pos, ce, 0.0))
    ce_neg = jnp.where(pos, 0.0, ce)

    ce_bits[:, :] = lax.bitcast_convert_type(ce_neg, jnp.int32)
    k_row = jnp.minimum(n_pos_row * _NEG_POS_RATIO, _P)

    def bs_body(_, carry):
        lo, hi = carry
        mid = lo + lax.div(hi - lo, 2)
        cnt = jnp.sum((ce_bits[:, :] >= mid).astype(jnp.int32), axis=1, keepdims=True)
        ge = cnt >= k_row
        return jnp.where(ge, mid, lo), jnp.where(ge, hi, mid)

    lo0 = jnp.zeros((_B, 1), jnp.int32)
    hi0 = jnp.full((_B, 1), _FLT_INF_BITS, jnp.int32)
    lo, _ = lax.fori_loop(0, 31, bs_body, (lo0, hi0))
    t = lax.bitcast_convert_type(lo, f32)
    gt = ce_neg > t
    cnt_gt = jnp.sum(gt.astype(f32), axis=1, keepdims=True)
    sum_gt = jnp.sum(jnp.where(gt, ce_neg, 0.0), axis=1, keepdims=True)
    neg_row = sum_gt + (k_row.astype(f32) - cnt_gt) * t
    neg_row = jnp.where(k_row == 0, 0.0, neg_row)
    neg_sum = jnp.sum(neg_row)

    conf = neg_sum + pos_sum
    contrib = (1.0 / _C) * (conf + _ALPHA * loc_loss) / n_pos_safe
    contrib = jnp.where(n_pos == 0.0, 0.0, contrib)
    out_ref[:, :] = out_ref[:, :] + jnp.reshape(contrib, (1, 1))


def kernel(predicted_locs, predicted_scores, boxes, labels, priors_cxcy):
    pl_t = jnp.transpose(predicted_locs, (1, 3, 0, 2))
    ps_t = jnp.transpose(predicted_scores, (1, 3, 0, 2))
    bx_t = jnp.transpose(boxes, (1, 3, 0, 2))
    lb_t = jnp.transpose(labels, (1, 0, 2))
    pr_t = jnp.transpose(priors_cxcy, (1, 0))

    out = pl.pallas_call(
        _mbl_kernel,
        grid=(_C,),
        in_specs=[
            pl.BlockSpec((1, 4, _B, _P), lambda c: (c, 0, 0, 0)),
            pl.BlockSpec((1, 2, _B, _P), lambda c: (c, 0, 0, 0)),
            pl.BlockSpec((1, 4, _B, _X), lambda c: (c, 0, 0, 0)),
            pl.BlockSpec((1, _B, _X), lambda c: (c, 0, 0)),
            pl.BlockSpec((4, _P), lambda c: (0, 0)),
        ],
        out_specs=pl.BlockSpec((1, 1), lambda c: (0, 0)),
        out_shape=jax.ShapeDtypeStruct((1, 1), jnp.float32),
        scratch_shapes=[pltpu.VMEM((_B, _P), jnp.int32)],
    )(pl_t, ps_t, bx_t, lb_t, pr_t)
    return out[0, 0]
